# baseline (speedup 1.0000x reference)
import functools

import jax
import jax.numpy as jnp
from jax import lax
from jax.experimental import pallas as pl
from jax.experimental.pallas import tpu as pltpu
from jax.experimental.pallas import tpu_sc as plsc

N = 10000
IN = 128
HID = 128
OUT = 64
V = 3
E = 320000
ALPHA = 0.5

NC = 2
NS = 16
NW = NC * NS
CHUNK = 64
EPT = E // NW
NCHUNK = 160
EPAD = NCHUNK * CHUNK
SUPC = 16
NSUP = NCHUNK // SUPC
ROWS_PER_TILE = 632
ACC_ROWS = NS * ROWS_PER_TILE
DEG_PER_TILE = 640
DEG_ROWS = NS * DEG_PER_TILE

def _sc_mesh():
    return plsc.VectorSubcoreMesh(core_axis_name="c", subcore_axis_name="s",
                                  num_cores=NC, num_subcores=NS)


def _elu(z):
    return jnp.where(z > 0, z, jnp.exp(jnp.minimum(z, 0.0)) - 1.0)


DEG_FLAT = V * DEG_ROWS
DEG_SLICE = DEG_FLAT // NS


def _sc_degree(dstp):
    @functools.partial(
        pl.kernel,
        out_type=jax.ShapeDtypeStruct((NC * DEG_FLAT,), jnp.float32),
        mesh=_sc_mesh(),
        scratch_types=[
            pltpu.VMEM_SHARED((DEG_FLAT,), jnp.float32),
            pltpu.VMEM((NCHUNK, CHUNK), jnp.int32),
            pltpu.VMEM((NCHUNK, CHUNK), jnp.int32),
            pltpu.VMEM((CHUNK,), jnp.float32),
            pltpu.VMEM((DEG_SLICE,), jnp.float32),
        ],
    )
    def k(dst_hbm, out_hbm, dacc, didx, didx2, ones, zbuf):
        c = lax.axis_index("c")
        s = lax.axis_index("s")
        w = s * NC + c

        def fill_ones(i, carry):
            ones[pl.ds(i * 16, 16)] = jnp.ones((16,), jnp.float32)
            return carry

        lax.fori_loop(0, CHUNK // 16, fill_ones, 0)

        def fill_zero(i, carry):
            zbuf[pl.ds(i * 16, 16)] = jnp.zeros((16,), jnp.float32)
            return carry

        lax.fori_loop(0, DEG_SLICE // 16, fill_zero, 0)

        pltpu.sync_copy(zbuf, dacc.at[pl.ds(s * DEG_SLICE, DEG_SLICE)])
        plsc.subcore_barrier()

        for v in range(V):
            pltpu.sync_copy(dst_hbm.at[v, w], didx)
            off = jnp.full((16,), v * DEG_ROWS, jnp.int32)

            def shift(j, carry):
                for kk in range(CHUNK // 16):
                    didx2[j, pl.ds(kk * 16, 16)] = \
                        didx[j, pl.ds(kk * 16, 16)] + off
                return carry

            lax.fori_loop(0, NCHUNK, shift, 0)

            def body(j, carry):
                pltpu.sync_copy(ones, dacc.at[didx2.at[j]], add=True)
                return carry

            lax.fori_loop(0, NCHUNK, body, 0)

        plsc.subcore_barrier()
        pltpu.sync_copy(
            dacc.at[pl.ds(s * DEG_SLICE, DEG_SLICE)],
            out_hbm.at[pl.ds(c * DEG_FLAT + s * DEG_SLICE, DEG_SLICE)],
        )

    return k(dstp).reshape(NC, V, DEG_ROWS)


def _sc_scatter_rows(y, srcf, dstp):
    @functools.partial(
        pl.kernel,
        out_type=jax.ShapeDtypeStruct((NC, ACC_ROWS, HID), jnp.float32),
        mesh=_sc_mesh(),
        scratch_types=[
            pltpu.VMEM_SHARED((ACC_ROWS, HID), jnp.float32),
            pltpu.VMEM((EPAD,), jnp.int32),
            pltpu.VMEM((SUPC, CHUNK), jnp.int32),
            pltpu.VMEM((SUPC, CHUNK), jnp.int32),
            pltpu.VMEM((SUPC, CHUNK), jnp.int32),
            pltpu.VMEM((CHUNK, HID), jnp.float32),
            pltpu.VMEM((CHUNK, HID), jnp.float32),
            pltpu.VMEM((CHUNK, HID), jnp.float32),
            pltpu.VMEM((CHUNK, HID), jnp.float32),
            pltpu.SemaphoreType.DMA,
            pltpu.SemaphoreType.DMA,
            pltpu.SemaphoreType.DMA,
            pltpu.SemaphoreType.DMA,
            pltpu.SemaphoreType.DMA,
            pltpu.SemaphoreType.DMA,
            pltpu.SemaphoreType.DMA,
            pltpu.SemaphoreType.DMA,
            pltpu.SemaphoreType.DMA,
            pltpu.SemaphoreType.DMA,
            pltpu.SemaphoreType.DMA,
            pltpu.SemaphoreType.DMA,
        ],
    )
    def k(y_hbm, src_hbm, dst_hbm, out_hbm, acc, sb, d0, d1, d2,
          b0, b1, b2, b3,
          sg0, sg1, sg2, sg3, ss0, ss1, ss2, ss3, ps, pd0, pd1, pd2):
        c = lax.axis_index("c")
        s = lax.axis_index("s")
        w = s * NC + c
        base = s * ROWS_PER_TILE
        bufs = (b0, b1, b2, b3)
        sg = (sg0, sg1, sg2, sg3)
        ss = (ss0, ss1, ss2, ss3)
        dbs = (d0, d1, d2)
        pd = (pd0, pd1, pd2)

        pltpu.async_copy(src_hbm.at[w], sb, ps)
        pltpu.async_copy(dst_hbm.at[w, pl.ds(0, SUPC)], d0, pd0)
        pltpu.async_copy(dst_hbm.at[w, pl.ds(SUPC, SUPC)], d1, pd1)

        def fill_zero(i, carry):
            for kk in range(HID // 16):
                b0[i, pl.ds(kk * 16, 16)] = jnp.zeros((16,), jnp.float32)
            return carry

        lax.fori_loop(0, 16, fill_zero, 0)

        def zero_acc(i, carry):
            pltpu.sync_copy(b0.at[pl.ds(0, 16)],
                            acc.at[pl.ds(base + i * 16, 16)])
            return carry

        lax.fori_loop(0, 39, zero_acc, 0)
        pltpu.sync_copy(b0.at[pl.ds(0, 8)], acc.at[pl.ds(base + 624, 8)])
        plsc.subcore_barrier()

        pltpu.make_async_copy(src_hbm.at[w], sb, ps).wait()

        pltpu.async_copy(y_hbm.at[sb.at[pl.ds(0, CHUNK)]], bufs[0], sg[0])
        pltpu.async_copy(y_hbm.at[sb.at[pl.ds(CHUNK, CHUNK)]],
                         bufs[1], sg[1])

        for g in range(NSUP):
            p = g % 3
            db = dbs[p]
            pltpu.make_async_copy(
                dst_hbm.at[w, pl.ds(g * SUPC, SUPC)], db, pd[p]).wait()

            def body(qq, carry):
                for t in range(4):
                    jl = 4 * qq + t
                    jg = g * SUPC + jl
                    pltpu.make_async_copy(
                        y_hbm.at[sb.at[pl.ds(jg * CHUNK, CHUNK)]],
                        bufs[t], sg[t]).wait()
                    pltpu.async_copy(bufs[t], acc.at[db.at[jl]], ss[t],
                                     add=True)
                    tb = (t + 2) % 4

                    def wait_sc():
                        pltpu.make_async_copy(
                            bufs[tb], acc.at[db.at[jl]], ss[tb]).wait()

                    def next_gather():
                        pltpu.async_copy(
                            y_hbm.at[sb.at[pl.ds((jg + 2) * CHUNK, CHUNK)]],
                            bufs[tb], sg[tb])

                    if g == 0 and t < 2:
                        @pl.when(qq > 0)
                        def _w():
                            wait_sc()
                        next_gather()
                    elif g == NSUP - 1 and t >= 2:
                        wait_sc()

                        @pl.when(qq < SUPC // 4 - 1)
                        def _i():
                            next_gather()
                    else:
                        wait_sc()
                        next_gather()
                return carry

            lax.fori_loop(0, SUPC // 4, body, 0)

            if g + 2 < NSUP:
                pn = (g + 2) % 3
                pltpu.async_copy(
                    dst_hbm.at[w, pl.ds((g + 2) * SUPC, SUPC)],
                    dbs[pn], pd[pn])

        pltpu.make_async_copy(bufs[2], acc.at[dbs[(NSUP - 1) % 3].at[0]],
                              ss[2]).wait()
        pltpu.make_async_copy(bufs[3], acc.at[dbs[(NSUP - 1) % 3].at[0]],
                              ss[3]).wait()

        plsc.subcore_barrier()
        pltpu.sync_copy(
            acc.at[pl.ds(base, ROWS_PER_TILE)],
            out_hbm.at[c].at[pl.ds(base, ROWS_PER_TILE)],
        )

    return k(y, srcf, dstp)


def _tc_pre(x, W1, degp):
    R = 1000
    d0 = degp[0].T
    d1 = degp[1].T

    def body(x_ref, w_ref, d0_ref, d1_ref, xw_ref, y0_ref, y1_ref, y2_ref,
             dinv_ref):
        xb = x_ref[...]
        deg = d0_ref[...] + d1_ref[...] + 1.0
        dinv = lax.rsqrt(deg)
        dinv_ref[...] = dinv
        y_refs = (y0_ref, y1_ref, y2_ref)
        for v in range(V):
            xw = lax.dot_general(xb, w_ref[v], (((1,), (1,)), ((), ())),
                                 preferred_element_type=jnp.float32)
            xw_ref[v] = xw
            y_refs[v][...] = xw * dinv[:, v:v + 1]

    return pl.pallas_call(
        body,
        grid=(N // R,),
        in_specs=[
            pl.BlockSpec((R, IN), lambda i: (i, 0)),
            pl.BlockSpec((V, HID, IN), lambda i: (0, 0, 0)),
            pl.BlockSpec((R, V), lambda i: (i, 0)),
            pl.BlockSpec((R, V), lambda i: (i, 0)),
        ],
        out_specs=[
            pl.BlockSpec((V, R, HID), lambda i: (0, i, 0)),
            pl.BlockSpec((R, HID), lambda i: (i, 0)),
            pl.BlockSpec((R, HID), lambda i: (i, 0)),
            pl.BlockSpec((R, HID), lambda i: (i, 0)),
            pl.BlockSpec((R, V), lambda i: (i, 0)),
        ],
        out_shape=[
            jax.ShapeDtypeStruct((V, N, HID), jnp.float32),
            jax.ShapeDtypeStruct((N, HID), jnp.float32),
            jax.ShapeDtypeStruct((N, HID), jnp.float32),
            jax.ShapeDtypeStruct((N, HID), jnp.float32),
            jax.ShapeDtypeStruct((N, V), jnp.float32),
        ],
    )(x, W1, d0, d1)


def _tc_attn(Ws, A, ba):
    D = Ws.shape[2]

    def body(w_ref, a_ref, ba_ref, out_ref):
        rows = []
        for v in range(V):
            WA = lax.dot_general(w_ref[v], a_ref[...],
                                 (((1,), (0,)), ((), ())),
                                 preferred_element_type=jnp.float32)
            row = []
            for u in range(V):
                row.append(jnp.sum(WA * w_ref[u]))
            rows.append(jnp.stack(row))
        M = jnp.stack(rows) + jnp.float32(HID) * ba_ref[0]
        M = M - jnp.max(M, axis=1, keepdims=True)
        eM = jnp.exp(M)
        out_ref[...] = eM / jnp.sum(eM, axis=1, keepdims=True)

    return pl.pallas_call(
        body,
        in_specs=[
            pl.BlockSpec((V, HID, D), lambda: (0, 0, 0)),
            pl.BlockSpec((D, D), lambda: (0, 0)),
            pl.BlockSpec(memory_space=pltpu.SMEM),
        ],
        out_specs=pl.BlockSpec((V, V), lambda: (0, 0)),
        out_shape=jax.ShapeDtypeStruct((V, V), jnp.float32),
    )(Ws, A, ba.reshape(1))


def _tc_mid(xw, dinv, parts, b1, attn, Wc1, bc1, W2):
    R = 1000

    def body(xw_ref, dinv_ref, p00, p01, p10, p11, p20, p21, b1_ref,
             attn_ref, wc1_ref, bc1_ref, w2_ref,
             xw2_ref, y20_ref, y21_ref, y22_ref):
        dinv = dinv_ref[...]
        p_refs = ((p00, p01), (p10, p11), (p20, p21))
        xws = [xw_ref[v] for v in range(V)]
        h = jnp.broadcast_to(bc1_ref[...][None, :], (R, HID))
        xrs = []
        for v in range(V):
            pv = p_refs[v][0][...] + p_refs[v][1][...]
            dv = dinv[:, v:v + 1]
            conv = dv * pv + (dv * dv) * xws[v] + b1_ref[v][None, :]
            xm = _elu(conv)
            xc = attn_ref[v, 0] * xws[0] + attn_ref[v, 1] * xws[1] \
                + attn_ref[v, 2] * xws[2]
            xr = (1.0 - ALPHA) + xm + _elu(ALPHA * xc)
            xrs.append(xr)
        for v in range(V):
            wc1_v = wc1_ref[:, v * HID:(v + 1) * HID]
            h = h + lax.dot_general(xrs[v], wc1_v, (((1,), (1,)), ((), ())),
                                    preferred_element_type=jnp.float32)
        h = _elu(h)
        y2_refs = (y20_ref, y21_ref, y22_ref)
        for v in range(V):
            xw2 = lax.dot_general(h, w2_ref[v], (((1,), (1,)), ((), ())),
                                  preferred_element_type=jnp.float32)
            xw2_ref[v] = xw2
            y2_refs[v][...] = xw2 * dinv[:, v:v + 1]

    part_spec = pl.BlockSpec((R, HID), lambda i: (i, 0))
    return pl.pallas_call(
        body,
        grid=(N // R,),
        in_specs=[
            pl.BlockSpec((V, R, HID), lambda i: (0, i, 0)),
            pl.BlockSpec((R, V), lambda i: (i, 0)),
            part_spec, part_spec, part_spec, part_spec, part_spec, part_spec,
            pl.BlockSpec((V, HID), lambda i: (0, 0)),
            pl.BlockSpec(memory_space=pltpu.SMEM),
            pl.BlockSpec((HID, V * HID), lambda i: (0, 0)),
            pl.BlockSpec((HID,), lambda i: (0,)),
            pl.BlockSpec((V, HID, HID), lambda i: (0, 0, 0)),
        ],
        out_specs=[
            pl.BlockSpec((V, R, HID), lambda i: (0, i, 0)),
            pl.BlockSpec((R, HID), lambda i: (i, 0)),
            pl.BlockSpec((R, HID), lambda i: (i, 0)),
            pl.BlockSpec((R, HID), lambda i: (i, 0)),
        ],
        out_shape=[
            jax.ShapeDtypeStruct((V, N, HID), jnp.float32),
            jax.ShapeDtypeStruct((N, HID), jnp.float32),
            jax.ShapeDtypeStruct((N, HID), jnp.float32),
            jax.ShapeDtypeStruct((N, HID), jnp.float32),
        ],
    )(xw, dinv, parts[0][0], parts[0][1], parts[1][0], parts[1][1],
      parts[2][0], parts[2][1], b1, attn, Wc1, bc1, W2)


def _tc_final(xw2, dinv, parts, b2, attn, Wc2, bc2):
    R = 1000

    def body(xw_ref, dinv_ref, p00, p01, p10, p11, p20, p21, b2_ref,
             attn_ref, wc2_ref, bc2_ref, out_ref):
        dinv = dinv_ref[...]
        p_refs = ((p00, p01), (p10, p11), (p20, p21))
        xws = [xw_ref[v] for v in range(V)]
        o = jnp.broadcast_to(bc2_ref[...][None, :], (R, OUT))
        for v in range(V):
            pv = p_refs[v][0][...] + p_refs[v][1][...]
            dv = dinv[:, v:v + 1]
            conv = dv * pv + (dv * dv) * xws[v] + b2_ref[v][None, :]
            xc = attn_ref[v, 0] * xws[0] + attn_ref[v, 1] * xws[1] \
                + attn_ref[v, 2] * xws[2]
            xr = (1.0 - ALPHA) + conv + _elu(ALPHA * xc)
            wc2_v = wc2_ref[:, v * HID:(v + 1) * HID]
            o = o + lax.dot_general(xr, wc2_v, (((1,), (1,)), ((), ())),
                                    preferred_element_type=jnp.float32)
        o = _elu(o)
        m = jnp.max(o, axis=1, keepdims=True)
        z = o - m
        lse = jnp.log(jnp.sum(jnp.exp(z), axis=1, keepdims=True))
        out_ref[...] = z - lse

    part_spec = pl.BlockSpec((R, HID), lambda i: (i, 0))
    return pl.pallas_call(
        body,
        grid=(N // R,),
        in_specs=[
            pl.BlockSpec((V, R, HID), lambda i: (0, i, 0)),
            pl.BlockSpec((R, V), lambda i: (i, 0)),
            part_spec, part_spec, part_spec, part_spec, part_spec, part_spec,
            pl.BlockSpec((V, HID), lambda i: (0, 0)),
            pl.BlockSpec(memory_space=pltpu.SMEM),
            pl.BlockSpec((OUT, V * HID), lambda i: (0, 0)),
            pl.BlockSpec((OUT,), lambda i: (0,)),
        ],
        out_specs=pl.BlockSpec((R, OUT), lambda i: (i, 0)),
        out_shape=jax.ShapeDtypeStruct((N, OUT), jnp.float32),
    )(xw2, dinv, parts[0][0], parts[0][1], parts[1][0], parts[1][1],
      parts[2][0], parts[2][1], b2, attn, Wc2, bc2)


def kernel(x, edge_index, W1, b1, W2, b2, A1, ba1, A2, ba2, Wc1, bc1,
           Wc2, bc2):
    pad = NW * EPAD - E
    src_pad = jnp.zeros((V, pad), jnp.int32)
    dst_pad = jnp.full((V, pad), N, jnp.int32)
    srcf = jnp.concatenate([edge_index[:, 0, :], src_pad], axis=1) \
        .reshape(V, NW, EPAD)
    dstp = jnp.concatenate([edge_index[:, 1, :], dst_pad], axis=1) \
        .reshape(V, NW, NCHUNK, CHUNK)

    degp = _sc_degree(dstp)
    xw, y0, y1, y2, dinv = _tc_pre(x, W1, degp)

    ys = (y0, y1, y2)
    parts1 = []
    for v in range(V):
        p = _sc_scatter_rows(ys[v], srcf[v], dstp[v])
        parts1.append((p[0, :N], p[1, :N]))

    attn1 = _tc_attn(W1, A1, ba1)
    xw2, y20, y21, y22 = _tc_mid(xw, dinv, parts1, b1, attn1, Wc1, bc1, W2)

    y2s = (y20, y21, y22)
    parts2 = []
    for v in range(V):
        p = _sc_scatter_rows(y2s[v], srcf[v], dstp[v])
        parts2.append((p[0, :N], p[1, :N]))

    attn2 = _tc_attn(W2, A2, ba2)
    return _tc_final(xw2, dinv, parts2, b2, attn2, Wc2, bc2)


# speedup vs baseline: 1.0372x; 1.0372x over previous
import functools

import jax
import jax.numpy as jnp
from jax import lax
from jax.experimental import pallas as pl
from jax.experimental.pallas import tpu as pltpu
from jax.experimental.pallas import tpu_sc as plsc

N = 10000
IN = 128
HID = 128
OUT = 64
V = 3
E = 320000
ALPHA = 0.5

NC = 2
NS = 16
NW = NC * NS
CHUNK = 64
EPT = E // NW
NCHUNK = 160
EPAD = NCHUNK * CHUNK
SUPC = 32
NSUP = NCHUNK // SUPC
QGRP = SUPC // 4
ROWS_PER_TILE = 632
ACC_ROWS = NS * ROWS_PER_TILE
DEG_PER_TILE = 640
DEG_ROWS = NS * DEG_PER_TILE

def _sc_mesh():
    return plsc.VectorSubcoreMesh(core_axis_name="c", subcore_axis_name="s",
                                  num_cores=NC, num_subcores=NS)


def _elu(z):
    return jnp.where(z > 0, z, jnp.exp(jnp.minimum(z, 0.0)) - 1.0)


DEG_FLAT = V * DEG_ROWS
DEG_SLICE = DEG_FLAT // NS


def _sc_degree(dstp):
    @functools.partial(
        pl.kernel,
        out_type=jax.ShapeDtypeStruct((NC * DEG_FLAT,), jnp.float32),
        mesh=_sc_mesh(),
        scratch_types=[
            pltpu.VMEM_SHARED((DEG_FLAT,), jnp.float32),
            pltpu.VMEM((NCHUNK, CHUNK), jnp.int32),
            pltpu.VMEM((NCHUNK, CHUNK), jnp.int32),
            pltpu.VMEM((CHUNK,), jnp.float32),
            pltpu.VMEM((DEG_SLICE,), jnp.float32),
        ],
    )
    def k(dst_hbm, out_hbm, dacc, didx, didx2, ones, zbuf):
        c = lax.axis_index("c")
        s = lax.axis_index("s")
        w = s * NC + c

        def fill_ones(i, carry):
            ones[pl.ds(i * 16, 16)] = jnp.ones((16,), jnp.float32)
            return carry

        lax.fori_loop(0, CHUNK // 16, fill_ones, 0)

        def fill_zero(i, carry):
            zbuf[pl.ds(i * 16, 16)] = jnp.zeros((16,), jnp.float32)
            return carry

        lax.fori_loop(0, DEG_SLICE // 16, fill_zero, 0)

        pltpu.sync_copy(zbuf, dacc.at[pl.ds(s * DEG_SLICE, DEG_SLICE)])
        plsc.subcore_barrier()

        for v in range(V):
            pltpu.sync_copy(dst_hbm.at[v, w], didx)
            off = jnp.full((16,), v * DEG_ROWS, jnp.int32)

            def shift(j, carry):
                for kk in range(CHUNK // 16):
                    didx2[j, pl.ds(kk * 16, 16)] = \
                        didx[j, pl.ds(kk * 16, 16)] + off
                return carry

            lax.fori_loop(0, NCHUNK, shift, 0)

            def body(j, carry):
                pltpu.sync_copy(ones, dacc.at[didx2.at[j]], add=True)
                return carry

            lax.fori_loop(0, NCHUNK, body, 0)

        plsc.subcore_barrier()
        pltpu.sync_copy(
            dacc.at[pl.ds(s * DEG_SLICE, DEG_SLICE)],
            out_hbm.at[pl.ds(c * DEG_FLAT + s * DEG_SLICE, DEG_SLICE)],
        )

    return k(dstp).reshape(NC, V, DEG_ROWS)


def _sc_scatter_rows(y, srcp, dstp):
    @functools.partial(
        pl.kernel,
        out_type=jax.ShapeDtypeStruct((NC, ACC_ROWS, HID), jnp.float32),
        mesh=_sc_mesh(),
        scratch_types=[
            pltpu.VMEM_SHARED((ACC_ROWS, HID), jnp.float32),
            pltpu.VMEM((SUPC, CHUNK), jnp.int32),
            pltpu.VMEM((SUPC, CHUNK), jnp.int32),
            pltpu.VMEM((SUPC, CHUNK), jnp.int32),
            pltpu.VMEM((SUPC, CHUNK), jnp.int32),
            pltpu.VMEM((CHUNK, HID), jnp.float32),
            pltpu.VMEM((CHUNK, HID), jnp.float32),
            pltpu.VMEM((CHUNK, HID), jnp.float32),
            pltpu.VMEM((CHUNK, HID), jnp.float32),
            pltpu.SemaphoreType.DMA,
            pltpu.SemaphoreType.DMA,
            pltpu.SemaphoreType.DMA,
            pltpu.SemaphoreType.DMA,
            pltpu.SemaphoreType.DMA,
            pltpu.SemaphoreType.DMA,
            pltpu.SemaphoreType.DMA,
            pltpu.SemaphoreType.DMA,
            pltpu.SemaphoreType.DMA,
            pltpu.SemaphoreType.DMA,
        ],
    )
    def k(y_hbm, src_hbm, dst_hbm, out_hbm, acc, si0, si1, di0, di1,
          b0, b1, b2, b3,
          sg0, sg1, sg2, sg3, ss0, ss1, ss2, ss3, pi0, pi1):
        c = lax.axis_index("c")
        s = lax.axis_index("s")
        w = s * NC + c
        base = s * ROWS_PER_TILE
        bufs = (b0, b1, b2, b3)
        sg = (sg0, sg1, sg2, sg3)
        ss = (ss0, ss1, ss2, ss3)
        sidx = (si0, si1)
        didx = (di0, di1)
        pidx = (pi0, pi1)

        def fill_zero(i, carry):
            for kk in range(HID // 16):
                b0[i, pl.ds(kk * 16, 16)] = jnp.zeros((16,), jnp.float32)
            return carry

        lax.fori_loop(0, 16, fill_zero, 0)

        def zero_acc(i, carry):
            pltpu.sync_copy(b0.at[pl.ds(0, 16)],
                            acc.at[pl.ds(base + i * 16, 16)])
            return carry

        lax.fori_loop(0, 39, zero_acc, 0)
        pltpu.sync_copy(b0.at[pl.ds(0, 8)], acc.at[pl.ds(base + 624, 8)])
        plsc.subcore_barrier()

        pltpu.async_copy(src_hbm.at[w, pl.ds(0, SUPC)], si0, pi0)
        pltpu.async_copy(dst_hbm.at[w, pl.ds(0, SUPC)], di0, pi0)

        for g in range(NSUP):
            p = g % 2
            sb, db = sidx[p], didx[p]
            pltpu.make_async_copy(
                src_hbm.at[w, pl.ds(g * SUPC, SUPC)], sb, pidx[p]).wait()
            pltpu.make_async_copy(
                dst_hbm.at[w, pl.ds(g * SUPC, SUPC)], db, pidx[p]).wait()
            if g + 1 < NSUP:
                pn = (g + 1) % 2
                pltpu.async_copy(
                    src_hbm.at[w, pl.ds((g + 1) * SUPC, SUPC)],
                    sidx[pn], pidx[pn])
                pltpu.async_copy(
                    dst_hbm.at[w, pl.ds((g + 1) * SUPC, SUPC)],
                    didx[pn], pidx[pn])

            pltpu.async_copy(y_hbm.at[sb.at[0]], bufs[0], sg[0])
            pltpu.async_copy(y_hbm.at[sb.at[1]], bufs[1], sg[1])

            def body(q, carry):
                for t in range(4):
                    j = 4 * q + t
                    pltpu.make_async_copy(
                        y_hbm.at[sb.at[j]], bufs[t], sg[t]).wait()
                    pltpu.async_copy(bufs[t], acc.at[db.at[j]], ss[t],
                                     add=True)
                    tb = (t + 2) % 4
                    if t < 2:
                        @pl.when(q > 0)
                        def _wait_sc():
                            pltpu.make_async_copy(
                                bufs[tb], acc.at[db.at[j]], ss[tb]).wait()
                        pltpu.async_copy(y_hbm.at[sb.at[j + 2]],
                                         bufs[tb], sg[tb])
                    else:
                        pltpu.make_async_copy(
                            bufs[tb], acc.at[db.at[j]], ss[tb]).wait()

                        @pl.when(q < QGRP - 1)
                        def _pref():
                            pltpu.async_copy(y_hbm.at[sb.at[j + 2]],
                                             bufs[tb], sg[tb])
                return carry

            lax.fori_loop(0, QGRP, body, 0)
            pltpu.make_async_copy(bufs[2], acc.at[db.at[0]], ss[2]).wait()
            pltpu.make_async_copy(bufs[3], acc.at[db.at[0]], ss[3]).wait()

        plsc.subcore_barrier()
        pltpu.sync_copy(
            acc.at[pl.ds(base, ROWS_PER_TILE)],
            out_hbm.at[c].at[pl.ds(base, ROWS_PER_TILE)],
        )

    return k(y, srcp, dstp)


def _tc_pre(x, W1, degp):
    R = 1000
    d0 = degp[0].T
    d1 = degp[1].T

    def body(x_ref, w_ref, d0_ref, d1_ref, xw_ref, y0_ref, y1_ref, y2_ref,
             dinv_ref):
        xb = x_ref[...]
        deg = d0_ref[...] + d1_ref[...] + 1.0
        dinv = lax.rsqrt(deg)
        dinv_ref[...] = dinv
        y_refs = (y0_ref, y1_ref, y2_ref)
        for v in range(V):
            xw = lax.dot_general(xb, w_ref[v], (((1,), (1,)), ((), ())),
                                 preferred_element_type=jnp.float32)
            xw_ref[v] = xw
            y_refs[v][...] = xw * dinv[:, v:v + 1]

    return pl.pallas_call(
        body,
        grid=(N // R,),
        in_specs=[
            pl.BlockSpec((R, IN), lambda i: (i, 0)),
            pl.BlockSpec((V, HID, IN), lambda i: (0, 0, 0)),
            pl.BlockSpec((R, V), lambda i: (i, 0)),
            pl.BlockSpec((R, V), lambda i: (i, 0)),
        ],
        out_specs=[
            pl.BlockSpec((V, R, HID), lambda i: (0, i, 0)),
            pl.BlockSpec((R, HID), lambda i: (i, 0)),
            pl.BlockSpec((R, HID), lambda i: (i, 0)),
            pl.BlockSpec((R, HID), lambda i: (i, 0)),
            pl.BlockSpec((R, V), lambda i: (i, 0)),
        ],
        out_shape=[
            jax.ShapeDtypeStruct((V, N, HID), jnp.float32),
            jax.ShapeDtypeStruct((N, HID), jnp.float32),
            jax.ShapeDtypeStruct((N, HID), jnp.float32),
            jax.ShapeDtypeStruct((N, HID), jnp.float32),
            jax.ShapeDtypeStruct((N, V), jnp.float32),
        ],
    )(x, W1, d0, d1)


def _tc_attn(Ws, A, ba):
    D = Ws.shape[2]

    def body(w_ref, a_ref, ba_ref, out_ref):
        rows = []
        for v in range(V):
            WA = lax.dot_general(w_ref[v], a_ref[...],
                                 (((1,), (0,)), ((), ())),
                                 preferred_element_type=jnp.float32)
            row = []
            for u in range(V):
                row.append(jnp.sum(WA * w_ref[u]))
            rows.append(jnp.stack(row))
        M = jnp.stack(rows) + jnp.float32(HID) * ba_ref[0]
        M = M - jnp.max(M, axis=1, keepdims=True)
        eM = jnp.exp(M)
        out_ref[...] = eM / jnp.sum(eM, axis=1, keepdims=True)

    return pl.pallas_call(
        body,
        in_specs=[
            pl.BlockSpec((V, HID, D), lambda: (0, 0, 0)),
            pl.BlockSpec((D, D), lambda: (0, 0)),
            pl.BlockSpec(memory_space=pltpu.SMEM),
        ],
        out_specs=pl.BlockSpec((V, V), lambda: (0, 0)),
        out_shape=jax.ShapeDtypeStruct((V, V), jnp.float32),
    )(Ws, A, ba.reshape(1))


def _tc_mid(xw, dinv, parts, b1, attn, Wc1, bc1, W2):
    R = 1000

    def body(xw_ref, dinv_ref, p00, p01, p10, p11, p20, p21, b1_ref,
             attn_ref, wc1_ref, bc1_ref, w2_ref,
             xw2_ref, y20_ref, y21_ref, y22_ref):
        dinv = dinv_ref[...]
        p_refs = ((p00, p01), (p10, p11), (p20, p21))
        xws = [xw_ref[v] for v in range(V)]
        h = jnp.broadcast_to(bc1_ref[...][None, :], (R, HID))
        xrs = []
        for v in range(V):
            pv = p_refs[v][0][...] + p_refs[v][1][...]
            dv = dinv[:, v:v + 1]
            conv = dv * pv + (dv * dv) * xws[v] + b1_ref[v][None, :]
            xm = _elu(conv)
            xc = attn_ref[v, 0] * xws[0] + attn_ref[v, 1] * xws[1] \
                + attn_ref[v, 2] * xws[2]
            xr = (1.0 - ALPHA) + xm + _elu(ALPHA * xc)
            xrs.append(xr)
        for v in range(V):
            wc1_v = wc1_ref[:, v * HID:(v + 1) * HID]
            h = h + lax.dot_general(xrs[v], wc1_v, (((1,), (1,)), ((), ())),
                                    preferred_element_type=jnp.float32)
        h = _elu(h)
        y2_refs = (y20_ref, y21_ref, y22_ref)
        for v in range(V):
            xw2 = lax.dot_general(h, w2_ref[v], (((1,), (1,)), ((), ())),
                                  preferred_element_type=jnp.float32)
            xw2_ref[v] = xw2
            y2_refs[v][...] = xw2 * dinv[:, v:v + 1]

    part_spec = pl.BlockSpec((R, HID), lambda i: (i, 0))
    return pl.pallas_call(
        body,
        grid=(N // R,),
        in_specs=[
            pl.BlockSpec((V, R, HID), lambda i: (0, i, 0)),
            pl.BlockSpec((R, V), lambda i: (i, 0)),
            part_spec, part_spec, part_spec, part_spec, part_spec, part_spec,
            pl.BlockSpec((V, HID), lambda i: (0, 0)),
            pl.BlockSpec(memory_space=pltpu.SMEM),
            pl.BlockSpec((HID, V * HID), lambda i: (0, 0)),
            pl.BlockSpec((HID,), lambda i: (0,)),
            pl.BlockSpec((V, HID, HID), lambda i: (0, 0, 0)),
        ],
        out_specs=[
            pl.BlockSpec((V, R, HID), lambda i: (0, i, 0)),
            pl.BlockSpec((R, HID), lambda i: (i, 0)),
            pl.BlockSpec((R, HID), lambda i: (i, 0)),
            pl.BlockSpec((R, HID), lambda i: (i, 0)),
        ],
        out_shape=[
            jax.ShapeDtypeStruct((V, N, HID), jnp.float32),
            jax.ShapeDtypeStruct((N, HID), jnp.float32),
            jax.ShapeDtypeStruct((N, HID), jnp.float32),
            jax.ShapeDtypeStruct((N, HID), jnp.float32),
        ],
    )(xw, dinv, parts[0][0], parts[0][1], parts[1][0], parts[1][1],
      parts[2][0], parts[2][1], b1, attn, Wc1, bc1, W2)


def _tc_final(xw2, dinv, parts, b2, attn, Wc2, bc2):
    R = 1000

    def body(xw_ref, dinv_ref, p00, p01, p10, p11, p20, p21, b2_ref,
             attn_ref, wc2_ref, bc2_ref, out_ref):
        dinv = dinv_ref[...]
        p_refs = ((p00, p01), (p10, p11), (p20, p21))
        xws = [xw_ref[v] for v in range(V)]
        o = jnp.broadcast_to(bc2_ref[...][None, :], (R, OUT))
        for v in range(V):
            pv = p_refs[v][0][...] + p_refs[v][1][...]
            dv = dinv[:, v:v + 1]
            conv = dv * pv + (dv * dv) * xws[v] + b2_ref[v][None, :]
            xc = attn_ref[v, 0] * xws[0] + attn_ref[v, 1] * xws[1] \
                + attn_ref[v, 2] * xws[2]
            xr = (1.0 - ALPHA) + conv + _elu(ALPHA * xc)
            wc2_v = wc2_ref[:, v * HID:(v + 1) * HID]
            o = o + lax.dot_general(xr, wc2_v, (((1,), (1,)), ((), ())),
                                    preferred_element_type=jnp.float32)
        o = _elu(o)
        m = jnp.max(o, axis=1, keepdims=True)
        z = o - m
        lse = jnp.log(jnp.sum(jnp.exp(z), axis=1, keepdims=True))
        out_ref[...] = z - lse

    part_spec = pl.BlockSpec((R, HID), lambda i: (i, 0))
    return pl.pallas_call(
        body,
        grid=(N // R,),
        in_specs=[
            pl.BlockSpec((V, R, HID), lambda i: (0, i, 0)),
            pl.BlockSpec((R, V), lambda i: (i, 0)),
            part_spec, part_spec, part_spec, part_spec, part_spec, part_spec,
            pl.BlockSpec((V, HID), lambda i: (0, 0)),
            pl.BlockSpec(memory_space=pltpu.SMEM),
            pl.BlockSpec((OUT, V * HID), lambda i: (0, 0)),
            pl.BlockSpec((OUT,), lambda i: (0,)),
        ],
        out_specs=pl.BlockSpec((R, OUT), lambda i: (i, 0)),
        out_shape=jax.ShapeDtypeStruct((N, OUT), jnp.float32),
    )(xw2, dinv, parts[0][0], parts[0][1], parts[1][0], parts[1][1],
      parts[2][0], parts[2][1], b2, attn, Wc2, bc2)


def kernel(x, edge_index, W1, b1, W2, b2, A1, ba1, A2, ba2, Wc1, bc1,
           Wc2, bc2):
    pad = NW * EPAD - E
    src_pad = jnp.zeros((V, pad), jnp.int32)
    dst_pad = jnp.full((V, pad), N, jnp.int32)
    srcp = jnp.concatenate([edge_index[:, 0, :], src_pad], axis=1) \
        .reshape(V, NW, NCHUNK, CHUNK)
    dstp = jnp.concatenate([edge_index[:, 1, :], dst_pad], axis=1) \
        .reshape(V, NW, NCHUNK, CHUNK)

    degp = _sc_degree(dstp)
    xw, y0, y1, y2, dinv = _tc_pre(x, W1, degp)

    ys = (y0, y1, y2)
    parts1 = []
    for v in range(V):
        p = _sc_scatter_rows(ys[v], srcp[v], dstp[v])
        parts1.append((p[0, :N], p[1, :N]))

    attn1 = _tc_attn(W1, A1, ba1)
    xw2, y20, y21, y22 = _tc_mid(xw, dinv, parts1, b1, attn1, Wc1, bc1, W2)

    y2s = (y20, y21, y22)
    parts2 = []
    for v in range(V):
        p = _sc_scatter_rows(y2s[v], srcp[v], dstp[v])
        parts2.append((p[0, :N], p[1, :N]))

    attn2 = _tc_attn(W2, A2, ba2)
    return _tc_final(xw2, dinv, parts2, b2, attn2, Wc2, bc2)
